# baseline (speedup 1.0000x reference)
import functools

import jax
import jax.numpy as jnp
from jax import lax
from jax.experimental import pallas as pl
from jax.experimental.pallas import tpu as pltpu
from jax.experimental.pallas import tpu_sc as plsc

N_SRC = 10000
N_SEG = 10000
E = 320000
D = 128

NC = 2
NS = 16
NW = NC * NS
EPW = E // NW
W = 80
NWIN = EPW // W
NBUF = 2
CW = 125
CNWIN = EPW // CW
N_ACC = 10112
RPT = N_ACC // NS

_MESH = plsc.VectorSubcoreMesh(core_axis_name="c", subcore_axis_name="s")


def _fill(ref, nrows, ncols, value):
    @pl.loop(0, nrows)
    def _(i):
        @pl.loop(0, ncols, step=16)
        def _(k):
            ref.at[pl.ds(i, 1), pl.ds(k, 16)][...] = jnp.full(
                (1, 16), value, jnp.float32)


def _zero_slice(src, dst, r0, w):
    nfull, rem = RPT // w, RPT % w

    @pl.loop(0, nfull)
    def _(k):
        pltpu.sync_copy(src, dst.at[pl.ds(r0 + k * w, w)])
    if rem:
        pltpu.sync_copy(src.at[pl.ds(0, rem)],
                        dst.at[pl.ds(r0 + nfull * w, rem)])


def _make_sums_kernel():
    @functools.partial(
        pl.kernel,
        mesh=_MESH,
        out_type=jax.ShapeDtypeStruct((NC, N_ACC, D), jnp.float32),
        scratch_types=[
            pltpu.VMEM((EPW,), jnp.int32),
            pltpu.VMEM((NWIN, W), jnp.int32),
        ] + [pltpu.VMEM((W, D), jnp.float32) for _ in range(NBUF)] + [
            pltpu.VMEM_SHARED((N_ACC, D), jnp.float32),
        ] + [pltpu.SemaphoreType.DMA for _ in range(NBUF)],
    )
    def sums_kernel(values_hbm, gidx_hbm, seg_hbm, sums_hbm,
                    gidx_v, seg_v, *rest):
        rows = rest[:NBUF]
        acc_sh = rest[NBUF]
        gsems = rest[NBUF + 1:]
        c = lax.axis_index("c")
        s = lax.axis_index("s")
        wid = s * NC + c

        pltpu.sync_copy(gidx_hbm.at[wid], gidx_v)
        pltpu.sync_copy(seg_hbm.at[wid], seg_v)

        _fill(rows[0], W, D, 0.0)
        r0 = s * RPT
        _zero_slice(rows[0], acc_sh, r0, W)
        plsc.subcore_barrier()

        def fire_gather(j, b):
            pltpu.async_copy(
                values_hbm.at[gidx_v.at[pl.ds(j * W, W)]], rows[b], gsems[b])

        def wait_gather(b):
            pltpu.make_async_copy(
                values_hbm.at[gidx_v.at[pl.ds(0, W)]], rows[b],
                gsems[b]).wait()

        def scatter(j, b):
            pltpu.sync_copy(rows[b], acc_sh.at[seg_v.at[j]], add=True)

        fire_gather(0, 0)
        fire_gather(1, 1)

        @pl.loop(0, NWIN - 3, step=2)
        def _(j0):
            wait_gather(0)
            scatter(j0, 0)
            fire_gather(j0 + 2, 0)
            wait_gather(1)
            scatter(j0 + 1, 1)
            fire_gather(j0 + 3, 1)

        wait_gather(0)
        scatter(NWIN - 3, 0)
        fire_gather(NWIN - 1, 0)
        wait_gather(1)
        scatter(NWIN - 2, 1)
        wait_gather(0)
        scatter(NWIN - 1, 0)

        plsc.subcore_barrier()
        pltpu.sync_copy(acc_sh.at[pl.ds(r0, RPT)],
                        sums_hbm.at[c, pl.ds(r0, RPT)])

    return sums_kernel


def _make_counts_kernel():
    @functools.partial(
        pl.kernel,
        mesh=_MESH,
        out_type=jax.ShapeDtypeStruct((NC, N_ACC, D), jnp.float32),
        scratch_types=[
            pltpu.VMEM((CNWIN, CW), jnp.int32),
            pltpu.VMEM((CW, D), jnp.float32),
            pltpu.VMEM((CW, D), jnp.float32),
            pltpu.VMEM_SHARED((N_ACC, D), jnp.float32),
        ],
    )
    def counts_kernel(seg_hbm, cnts_hbm, seg_v, ones_v, zb, cnt_sh):
        c = lax.axis_index("c")
        s = lax.axis_index("s")
        wid = s * NC + c

        pltpu.sync_copy(seg_hbm.at[wid], seg_v)
        _fill(ones_v, CW, D, 1.0)
        _fill(zb, CW, D, 0.0)
        r0 = s * RPT
        _zero_slice(zb, cnt_sh, r0, CW)
        plsc.subcore_barrier()

        @pl.loop(0, CNWIN)
        def _(j):
            pltpu.sync_copy(ones_v, cnt_sh.at[seg_v.at[j]], add=True)

        plsc.subcore_barrier()
        pltpu.sync_copy(cnt_sh.at[pl.ds(r0, RPT)],
                        cnts_hbm.at[c, pl.ds(r0, RPT)])

    return counts_kernel


_sums_kernel = _make_sums_kernel()
_counts_kernel = _make_counts_kernel()

BR = 1000


def _merge_body(s_ref, c_ref, o_ref):
    ssum = s_ref[0] + s_ref[1]
    cnt = c_ref[0, :, 0:1] + c_ref[1, :, 0:1]
    o_ref[...] = ssum / jnp.maximum(cnt, 1.0)


_merge = pl.pallas_call(
    _merge_body,
    grid=(N_SEG // BR,),
    in_specs=[
        pl.BlockSpec((NC, BR, D), lambda i: (0, i, 0)),
        pl.BlockSpec((NC, BR, D), lambda i: (0, i, 0)),
    ],
    out_specs=pl.BlockSpec((BR, D), lambda i: (i, 0)),
    out_shape=jax.ShapeDtypeStruct((N_SEG, D), jnp.float32),
)


@jax.jit
def _impl(values, gather_idx, segment_ids):
    g = gather_idx.reshape(NW, EPW)
    sg = segment_ids.reshape(NW, NWIN, W)
    csg = segment_ids.reshape(NW, CNWIN, CW)
    sums = _sums_kernel(values, g, sg)
    cnts = _counts_kernel(csg)
    return _merge(sums, cnts)


def kernel(values, gather_idx, segment_ids):
    return _impl(values, gather_idx, segment_ids)


# speedup vs baseline: 1.0170x; 1.0170x over previous
import functools

import jax
import jax.numpy as jnp
from jax import lax
from jax.experimental import pallas as pl
from jax.experimental.pallas import tpu as pltpu
from jax.experimental.pallas import tpu_sc as plsc

N_SRC = 10000
N_SEG = 10000
E = 320000
D = 128

NC = 2
NS = 16
NW = NC * NS
EPW = E // NW
W = 80
NWIN = EPW // W
NBUF = 2
N_ACC = 10112
RPT = N_ACC // NS

_MESH = plsc.VectorSubcoreMesh(core_axis_name="c", subcore_axis_name="s")


def _fill(ref, nrows, ncols, value):
    @pl.loop(0, nrows)
    def _(i):
        @pl.loop(0, ncols, step=16)
        def _(k):
            ref.at[pl.ds(i, 1), pl.ds(k, 16)][...] = jnp.full(
                (1, 16), value, jnp.float32)


def _zero_slice(src, dst, r0, w):
    nfull, rem = RPT // w, RPT % w

    @pl.loop(0, nfull)
    def _(k):
        pltpu.sync_copy(src, dst.at[pl.ds(r0 + k * w, w)])
    if rem:
        pltpu.sync_copy(src.at[pl.ds(0, rem)],
                        dst.at[pl.ds(r0 + nfull * w, rem)])


def _make_agg_kernel():
    @functools.partial(
        pl.kernel,
        mesh=_MESH,
        out_type=(jax.ShapeDtypeStruct((NC, N_ACC, D), jnp.float32),
                  jax.ShapeDtypeStruct((NC, N_ACC, D), jnp.float32)),
        scratch_types=[
            pltpu.VMEM((EPW,), jnp.int32),
            pltpu.VMEM((NWIN, W), jnp.int32),
        ] + [pltpu.VMEM((W, D), jnp.float32) for _ in range(NBUF)] + [
            pltpu.VMEM_SHARED((N_ACC, D), jnp.float32),
        ] + [pltpu.SemaphoreType.DMA for _ in range(NBUF)],
    )
    def agg_kernel(values_hbm, gidx_hbm, seg_hbm, sums_hbm, cnts_hbm,
                   gidx_v, seg_v, *rest):
        rows = rest[:NBUF]
        acc_sh = rest[NBUF]
        gsems = rest[NBUF + 1:]
        c = lax.axis_index("c")
        s = lax.axis_index("s")
        wid = s * NC + c

        pltpu.sync_copy(gidx_hbm.at[wid], gidx_v)
        pltpu.sync_copy(seg_hbm.at[wid], seg_v)

        _fill(rows[0], W, D, 0.0)
        r0 = s * RPT
        _zero_slice(rows[0], acc_sh, r0, W)
        plsc.subcore_barrier()

        def fire_gather(j, b):
            pltpu.async_copy(
                values_hbm.at[gidx_v.at[pl.ds(j * W, W)]], rows[b], gsems[b])

        def wait_gather(b):
            pltpu.make_async_copy(
                values_hbm.at[gidx_v.at[pl.ds(0, W)]], rows[b],
                gsems[b]).wait()

        def scatter(j, b):
            pltpu.sync_copy(rows[b], acc_sh.at[seg_v.at[j]], add=True)

        fire_gather(0, 0)
        fire_gather(1, 1)

        @pl.loop(0, NWIN - 3, step=2)
        def _(j0):
            wait_gather(0)
            scatter(j0, 0)
            fire_gather(j0 + 2, 0)
            wait_gather(1)
            scatter(j0 + 1, 1)
            fire_gather(j0 + 3, 1)

        wait_gather(0)
        scatter(NWIN - 3, 0)
        fire_gather(NWIN - 1, 0)
        wait_gather(1)
        scatter(NWIN - 2, 1)
        wait_gather(0)
        scatter(NWIN - 1, 0)

        plsc.subcore_barrier()
        pltpu.sync_copy(acc_sh.at[pl.ds(r0, RPT)],
                        sums_hbm.at[c, pl.ds(r0, RPT)])

        _fill(rows[0], W, D, 0.0)
        _fill(rows[1], W, D, 1.0)
        _zero_slice(rows[0], acc_sh, r0, W)
        plsc.subcore_barrier()

        @pl.loop(0, NWIN)
        def _(j):
            pltpu.sync_copy(rows[1], acc_sh.at[seg_v.at[j]], add=True)

        plsc.subcore_barrier()
        pltpu.sync_copy(acc_sh.at[pl.ds(r0, RPT)],
                        cnts_hbm.at[c, pl.ds(r0, RPT)])

    return agg_kernel


_agg_kernel = _make_agg_kernel()

BR = 1000


def _merge_body(s_ref, c_ref, o_ref):
    ssum = s_ref[0] + s_ref[1]
    cnt = c_ref[0, :, 0:1] + c_ref[1, :, 0:1]
    o_ref[...] = ssum / jnp.maximum(cnt, 1.0)


_merge = pl.pallas_call(
    _merge_body,
    grid=(N_SEG // BR,),
    in_specs=[
        pl.BlockSpec((NC, BR, D), lambda i: (0, i, 0)),
        pl.BlockSpec((NC, BR, D), lambda i: (0, i, 0)),
    ],
    out_specs=pl.BlockSpec((BR, D), lambda i: (i, 0)),
    out_shape=jax.ShapeDtypeStruct((N_SEG, D), jnp.float32),
)


@jax.jit
def _impl(values, gather_idx, segment_ids):
    g = gather_idx.reshape(NW, EPW)
    sg = segment_ids.reshape(NW, NWIN, W)
    sums, cnts = _agg_kernel(values, g, sg)
    return _merge(sums, cnts)


def kernel(values, gather_idx, segment_ids):
    return _impl(values, gather_idx, segment_ids)
